# Initial kernel scaffold; baseline (speedup 1.0000x reference)
import jax
import jax.numpy as jnp
from jax.experimental import pallas as pl


def kernel(x, batch):
    raise NotImplementedError("write your pallas kernel here")


# speedup vs baseline: 3.3964x; 3.3964x over previous
import functools

import jax
import jax.numpy as jnp
from jax import lax
from jax.experimental import pallas as pl
from jax.experimental.pallas import tpu as pltpu
from jax.experimental.pallas import tpu_sc as plsc

N = 50000
D = 256
S = 128
NC = 2
NS = 16
L = 16
DC = D // NC
C = 400
NCHUNK = N // C
ROUNDS = (NCHUNK + NS - 1) // NS
SEGT = S // NS


def _body(x_hbm, b_hbm, out_hbm, xbuf, ibuf, acc, cnt, shsum, shcnt, p2buf, cbuf):
    c = lax.axis_index("c")
    s = lax.axis_index("s")
    lane = lax.iota(jnp.int32, L)
    ones = jnp.ones((L,), jnp.float32)
    zf = jnp.zeros((L,), jnp.float32)
    mask0 = lane == 0

    def zero_body(i, _):
        for j in range(DC // L):
            acc[i, pl.ds(j * L, L)] = zf
        return 0

    lax.fori_loop(0, S, zero_body, 0)
    for j in range(S // L):
        cnt[pl.ds(j * L, L)] = zf

    def chunk_body(k, _):
        cid = k * NS + s

        @pl.when(cid < NCHUNK)
        def _():
            row0 = cid * C
            pltpu.sync_copy(x_hbm.at[pl.ds(row0, C), pl.ds(c * DC, DC)], xbuf)
            pltpu.sync_copy(b_hbm.at[pl.ds(row0, C)], ibuf)

            def group_body(g, _):
                sv = ibuf[pl.ds(g * L, L)]
                for r in range(L):
                    segv = sv.at[jnp.full((L,), r, jnp.int32)].get(
                        mode="promise_in_bounds"
                    )
                    plsc.addupdate_scatter(cnt, [segv], ones, mask=mask0)
                    for j in range(DC // L):
                        xv = xbuf[g * L + r, pl.ds(j * L, L)]
                        plsc.addupdate_scatter(acc, [segv, lane + j * L], xv)
                return 0

            lax.fori_loop(0, C // L, group_body, 0)

        return 0

    lax.fori_loop(0, ROUNDS, chunk_body, 0)

    pltpu.sync_copy(acc, shsum.at[s])
    pltpu.sync_copy(cnt, shcnt.at[s])
    plsc.subcore_barrier()

    for t in range(NS):
        pltpu.sync_copy(shsum.at[t, pl.ds(s * SEGT, SEGT)], p2buf.at[t])
    pltpu.sync_copy(shcnt, cbuf)

    cbase = (s // 2) * L
    cv = cbuf[0, pl.ds(cbase, L)]
    for t in range(1, NS):
        cv = cv + cbuf[t, pl.ds(cbase, L)]
    rv = 1.0 / jnp.maximum(cv, 1.0)

    def fin_body(i, _):
        lane_i = (s % 2) * SEGT + i
        rvec = rv.at[jnp.full((L,), lane_i, jnp.int32)].get(
            mode="promise_in_bounds"
        )
        for j in range(DC // L):
            v = p2buf[0, i, pl.ds(j * L, L)]
            for t in range(1, NS):
                v = v + p2buf[t, i, pl.ds(j * L, L)]
            p2buf[0, i, pl.ds(j * L, L)] = v * rvec
        return 0

    lax.fori_loop(0, SEGT, fin_body, 0)
    pltpu.sync_copy(p2buf.at[0], out_hbm.at[pl.ds(s * SEGT, SEGT), pl.ds(c * DC, DC)])


@functools.cache
def _build():
    mesh = plsc.VectorSubcoreMesh(
        core_axis_name="c", subcore_axis_name="s", num_cores=NC, num_subcores=NS
    )
    return pl.kernel(
        _body,
        out_type=jax.ShapeDtypeStruct((S, D), jnp.float32),
        mesh=mesh,
        compiler_params=pltpu.CompilerParams(needs_layout_passes=False),
        scratch_types=[
        pltpu.VMEM((C, DC), jnp.float32),
        pltpu.VMEM((C,), jnp.int32),
        pltpu.VMEM((S, DC), jnp.float32),
        pltpu.VMEM((S,), jnp.float32),
        pltpu.VMEM_SHARED((NS, S, DC), jnp.float32),
        pltpu.VMEM_SHARED((NS, S), jnp.float32),
        pltpu.VMEM((NS, SEGT, DC), jnp.float32),
        pltpu.VMEM((NS, S), jnp.float32),
        ],
    )


@jax.jit
def kernel(x, batch):
    return _build()(x, batch.astype(jnp.int32))


# speedup vs baseline: 5.0686x; 1.4923x over previous
import functools

import jax
import jax.numpy as jnp
from jax import lax
from jax.experimental import pallas as pl
from jax.experimental.pallas import tpu as pltpu
from jax.experimental.pallas import tpu_sc as plsc

N = 50000
D = 256
S = 128
NC = 2
NS = 16
L = 16
DC = D // NC
CF = 128
NF = N // CF
FR = NF // NS
EX = NF - FR * NS
TAIL = N - NF * CF
TROW = NF * CF
SEGT = S // NS


def _body(x_hbm, b_hbm, out_hbm, xbufs, ibufs, ibuft, cnt, p2buf, cbuf,
          shsum, shcnt):
    c = lax.axis_index("c")
    s = lax.axis_index("s")
    lane = lax.iota(jnp.int32, L)
    ones = jnp.ones((L,), jnp.float32)
    zf = jnp.zeros((L,), jnp.float32)
    mask0 = lane == 0

    def zero_body(i, _):
        for j in range(DC // L):
            xbufs[0, i, pl.ds(j * L, L)] = zf
        return 0

    lax.fori_loop(0, CF, zero_body, 0)
    pltpu.sync_copy(xbufs.at[0], shsum.at[s])
    for j in range(S // L):
        cnt[pl.ds(j * L, L)] = zf

    def count_rows(k, nrows):
        def group_body(g, _):
            sv = ibufs[k, 0, pl.ds(g * L, L)]
            for r in range(L):
                segv = sv.at[jnp.full((L,), r, jnp.int32)].get(
                    mode="promise_in_bounds")
                plsc.addupdate_scatter(cnt, [segv], ones, mask=mask0)
            return 0

        lax.fori_loop(0, nrows // L, group_body, 0)

    def do_chunk(k, cid, buf):
        row0 = cid * CF
        pltpu.sync_copy(b_hbm.at[pl.ds(row0, CF)], ibufs.at[k, 0])
        pltpu.sync_copy(
            x_hbm.at[pl.ds(row0, CF), pl.ds(c * DC, DC)], xbufs.at[buf])
        pltpu.sync_copy(
            xbufs.at[buf], shsum.at[s].at[ibufs.at[k, 0]], add=True)
        count_rows(k, CF)

    for k in range(FR):
        do_chunk(k, k * NS + s, k % 2)

    @pl.when(s < EX)
    def _():
        do_chunk(FR, FR * NS + s, FR % 2)

    @pl.when(s == EX)
    def _():
        pltpu.sync_copy(b_hbm.at[pl.ds(TROW, TAIL)], ibuft)
        pltpu.sync_copy(
            x_hbm.at[pl.ds(TROW, TAIL), pl.ds(c * DC, DC)],
            xbufs.at[FR % 2, pl.ds(0, TAIL)])
        pltpu.sync_copy(
            xbufs.at[FR % 2, pl.ds(0, TAIL)], shsum.at[s].at[ibuft], add=True)

        def tgroup(g, _):
            sv = ibuft[pl.ds(g * L, L)]
            for r in range(L):
                segv = sv.at[jnp.full((L,), r, jnp.int32)].get(
                    mode="promise_in_bounds")
                plsc.addupdate_scatter(cnt, [segv], ones, mask=mask0)
            return 0

        lax.fori_loop(0, TAIL // L, tgroup, 0)

    pltpu.sync_copy(cnt, shcnt.at[s])
    plsc.subcore_barrier()

    for t in range(NS):
        pltpu.sync_copy(shsum.at[t, pl.ds(s * SEGT, SEGT)], p2buf.at[t])
    pltpu.sync_copy(shcnt, cbuf)

    cbase = (s // 2) * L
    cv = cbuf[0, pl.ds(cbase, L)]
    for t in range(1, NS):
        cv = cv + cbuf[t, pl.ds(cbase, L)]
    rv = 1.0 / jnp.maximum(cv, 1.0)

    def fin_body(i, _):
        lane_i = (s % 2) * SEGT + i
        rvec = rv.at[jnp.full((L,), lane_i, jnp.int32)].get(
            mode="promise_in_bounds")
        for j in range(DC // L):
            v = p2buf[0, i, pl.ds(j * L, L)]
            for t in range(1, NS):
                v = v + p2buf[t, i, pl.ds(j * L, L)]
            p2buf[0, i, pl.ds(j * L, L)] = v * rvec
        return 0

    lax.fori_loop(0, SEGT, fin_body, 0)
    pltpu.sync_copy(p2buf.at[0], out_hbm.at[pl.ds(s * SEGT, SEGT), pl.ds(c * DC, DC)])


@functools.cache
def _build():
    mesh = plsc.VectorSubcoreMesh(
        core_axis_name="c", subcore_axis_name="s", num_cores=NC, num_subcores=NS
    )
    return pl.kernel(
        _body,
        out_type=jax.ShapeDtypeStruct((S, D), jnp.float32),
        mesh=mesh,
        compiler_params=pltpu.CompilerParams(needs_layout_passes=False),
        scratch_types=[
            pltpu.VMEM((2, CF, DC), jnp.float32),
            pltpu.VMEM((FR + 1, 1, CF), jnp.int32),
            pltpu.VMEM((TAIL,), jnp.int32),
            pltpu.VMEM((S,), jnp.float32),
            pltpu.VMEM((NS, SEGT, DC), jnp.float32),
            pltpu.VMEM((NS, S), jnp.float32),
            pltpu.VMEM_SHARED((NS, S, DC), jnp.float32),
            pltpu.VMEM_SHARED((NS, S), jnp.float32),
        ],
    )


@jax.jit
def kernel(x, batch):
    return _build()(x, batch.astype(jnp.int32))


# speedup vs baseline: 7.3580x; 1.4517x over previous
import functools

import jax
import jax.numpy as jnp
from jax import lax
from jax.experimental import pallas as pl
from jax.experimental.pallas import tpu as pltpu
from jax.experimental.pallas import tpu_sc as plsc

N = 50000
D = 256
S = 128
NC = 2
NS = 16
L = 16
DC = D // NC
CF = 128
NF = N // CF
FR = NF // NS
EX = NF - FR * NS
TAIL = N - NF * CF
TROW = NF * CF
SEGT = S // NS


def _body(x_hbm, b_hbm, out_hbm, xbufs, ibufs, ibuft, cnt, p2buf, cbuf,
          shsum, shcnt, sem_g0, sem_g1, sem_s0, sem_s1):
    c = lax.axis_index("c")
    s = lax.axis_index("s")
    lane = lax.iota(jnp.int32, L)
    ones = jnp.ones((L,), jnp.float32)
    zf = jnp.zeros((L,), jnp.float32)
    mask0 = lane == 0

    def zero_body(i, _):
        for j in range(DC // L):
            xbufs[0, i, pl.ds(j * L, L)] = zf
        return 0

    lax.fori_loop(0, CF, zero_body, 0)
    pltpu.sync_copy(xbufs.at[0], shsum.at[s])
    for j in range(S // L):
        cnt[pl.ds(j * L, L)] = zf

    def count_rows(k, nrows):
        def group_body(g, _):
            sv = ibufs[k, 0, pl.ds(g * L, L)]
            for r in range(L):
                segv = sv.at[jnp.full((L,), r, jnp.int32)].get(
                    mode="promise_in_bounds")
                plsc.addupdate_scatter(cnt, [segv], ones, mask=mask0)
            return 0

        lax.fori_loop(0, nrows // L, group_body, 0)

    sem_g = (sem_g0, sem_g1)
    sem_s = (sem_s0, sem_s1)
    sdesc = {}
    for k in range(FR):
        buf = k % 2
        if k >= 2:
            sdesc.pop(k - 2).wait()
        cid = k * NS + s
        row0 = cid * CF
        gd = pltpu.async_copy(
            x_hbm.at[pl.ds(row0, CF), pl.ds(c * DC, DC)], xbufs.at[buf],
            sem_g[buf])
        pltpu.sync_copy(b_hbm.at[pl.ds(row0, CF)], ibufs.at[k, 0])
        count_rows(k, CF)
        gd.wait()
        sdesc[k] = pltpu.async_copy(
            xbufs.at[buf], shsum.at[s].at[ibufs.at[k, 0]], sem_s[buf],
            add=True)

    sdesc.pop(FR - 2).wait()

    @pl.when(s < EX)
    def _():
        cid = FR * NS + s
        row0 = cid * CF
        pltpu.sync_copy(b_hbm.at[pl.ds(row0, CF)], ibufs.at[FR, 0])
        pltpu.sync_copy(
            x_hbm.at[pl.ds(row0, CF), pl.ds(c * DC, DC)], xbufs.at[FR % 2])
        pltpu.sync_copy(
            xbufs.at[FR % 2], shsum.at[s].at[ibufs.at[FR, 0]], add=True)
        count_rows(FR, CF)

    @pl.when(s == EX)
    def _():
        pltpu.sync_copy(b_hbm.at[pl.ds(TROW, TAIL)], ibuft)
        pltpu.sync_copy(
            x_hbm.at[pl.ds(TROW, TAIL), pl.ds(c * DC, DC)],
            xbufs.at[FR % 2, pl.ds(0, TAIL)])
        pltpu.sync_copy(
            xbufs.at[FR % 2, pl.ds(0, TAIL)], shsum.at[s].at[ibuft], add=True)

        def tgroup(g, _):
            sv = ibuft[pl.ds(g * L, L)]
            for r in range(L):
                segv = sv.at[jnp.full((L,), r, jnp.int32)].get(
                    mode="promise_in_bounds")
                plsc.addupdate_scatter(cnt, [segv], ones, mask=mask0)
            return 0

        lax.fori_loop(0, TAIL // L, tgroup, 0)

    sdesc.pop(FR - 1).wait()

    pltpu.sync_copy(cnt, shcnt.at[s])
    plsc.subcore_barrier()

    for t in range(NS):
        pltpu.sync_copy(shsum.at[t, pl.ds(s * SEGT, SEGT)], p2buf.at[t])
    pltpu.sync_copy(shcnt, cbuf)

    cbase = (s // 2) * L
    cv = cbuf[0, pl.ds(cbase, L)]
    for t in range(1, NS):
        cv = cv + cbuf[t, pl.ds(cbase, L)]
    rv = 1.0 / jnp.maximum(cv, 1.0)

    def fin_body(i, _):
        lane_i = (s % 2) * SEGT + i
        rvec = rv.at[jnp.full((L,), lane_i, jnp.int32)].get(
            mode="promise_in_bounds")
        for j in range(DC // L):
            v = p2buf[0, i, pl.ds(j * L, L)]
            for t in range(1, NS):
                v = v + p2buf[t, i, pl.ds(j * L, L)]
            p2buf[0, i, pl.ds(j * L, L)] = v * rvec
        return 0

    lax.fori_loop(0, SEGT, fin_body, 0)
    pltpu.sync_copy(p2buf.at[0], out_hbm.at[pl.ds(s * SEGT, SEGT), pl.ds(c * DC, DC)])


@functools.cache
def _build():
    mesh = plsc.VectorSubcoreMesh(
        core_axis_name="c", subcore_axis_name="s", num_cores=NC, num_subcores=NS
    )
    return pl.kernel(
        _body,
        out_type=jax.ShapeDtypeStruct((S, D), jnp.float32),
        mesh=mesh,
        compiler_params=pltpu.CompilerParams(needs_layout_passes=False),
        scratch_types=[
            pltpu.VMEM((2, CF, DC), jnp.float32),
            pltpu.VMEM((FR + 1, 1, CF), jnp.int32),
            pltpu.VMEM((TAIL,), jnp.int32),
            pltpu.VMEM((S,), jnp.float32),
            pltpu.VMEM((NS, SEGT, DC), jnp.float32),
            pltpu.VMEM((NS, S), jnp.float32),
            pltpu.VMEM_SHARED((NS, S, DC), jnp.float32),
            pltpu.VMEM_SHARED((NS, S), jnp.float32),
            pltpu.SemaphoreType.DMA,
            pltpu.SemaphoreType.DMA,
            pltpu.SemaphoreType.DMA,
            pltpu.SemaphoreType.DMA,
        ],
    )


@jax.jit
def kernel(x, batch):
    return _build()(x, batch.astype(jnp.int32))
